# baseline (speedup 1.0000x reference)
import functools

import jax
import jax.numpy as jnp
from jax import lax
from jax.experimental import pallas as pl
from jax.experimental.pallas import tpu as pltpu
from jax.experimental.pallas import tpu_sc as plsc

_BATCH = 16384
_HIST = 200
_EMBED = 64
_B = _BATCH * _HIST

_NC = 2
_NS = 16
_NW = _NC * _NS
_CH = 256
_NCHUNK = _B // (_NW * _CH)
_BT = _CH // 128

_S_E8 = 129
_S_BT = 10 * 129
_S_ET = _BT * 10 * 129
_TBUF = 8 * _S_ET // 8 * 8
_TBUF_WORDS = 8 * _S_ET

_mesh = plsc.VectorSubcoreMesh(core_axis_name="c", subcore_axis_name="s")


@functools.partial(
    pl.kernel,
    out_type=jax.ShapeDtypeStruct((_HIST, 8, 128, 8, 128), jnp.float32),
    mesh=_mesh,
    scratch_types=[
        pltpu.VMEM((_CH,), jnp.int32),
        pltpu.VMEM((_CH,), jnp.int32),
        pltpu.VMEM((_CH, _EMBED), jnp.float32),
        pltpu.VMEM((_CH, _EMBED), jnp.float32),
        pltpu.VMEM((8, _BT, 10, _S_E8), jnp.float32),
        pltpu.VMEM((8, _BT, 10, _S_E8), jnp.float32),
        pltpu.SemaphoreType.DMA,
        pltpu.SemaphoreType.DMA,
        pltpu.SemaphoreType.DMA,
        pltpu.SemaphoreType.DMA,
    ],
    compiler_params=pltpu.CompilerParams(
        use_tc_tiling_on_sc=False, needs_layout_passes=False
    ),
)
def _gather_kernel(codes_hbm, table_hbm, out_hbm, idx0, idx1, rows0, rows1,
                   tbufA, tbufB, gsem0, gsem1, ssemA, ssemB):
    wid = lax.axis_index("s") * _NC + lax.axis_index("c")
    ubase = wid * _NCHUNK
    iota = lax.iota(jnp.int32, 16)
    et_vecs, e8_vecs = [], []
    for e0 in (0, 16, 32, 48):
        e_vec = e0 + iota
        et_vecs.append(e_vec >> 3)
        e8_vecs.append(e_vec & 7)

    def start_gather(t, idx_v, rows_v, gsem):
        off = (ubase + t) * _CH
        pltpu.sync_copy(codes_hbm.at[pl.ds(off, _CH)], idx_v)
        pltpu.async_copy(table_hbm.at[idx_v], rows_v, gsem)

    def transpose_store(t, idx_v, rows_v, gsem, tbuf, ssem, store_outstanding):
        u = ubase + t
        h = u // (_BATCH // _CH)
        bt0 = (u % (_BATCH // _CH)) * _BT
        pltpu.make_async_copy(table_hbm.at[idx_v], rows_v, gsem).wait()

        @pl.when(store_outstanding)
        def _():
            pltpu.make_async_copy(
                tbuf.at[:, :, :8, :128],
                out_hbm.at[h, :, pl.ds(bt0, _BT), :, :], ssem,
            ).wait()

        for btl in range(_BT):
            btl_vec = jnp.full((16,), btl, jnp.int32)

            @plsc.parallel_loop(0, 128, unroll=4)
            def _row(b, btl=btl, btl_vec=btl_vec):
                r = btl * 128 + b
                b_vec = jnp.full((16,), b, jnp.int32)
                for k in range(4):
                    v = rows_v[r, pl.ds(k * 16, 16)]
                    plsc.store_scatter(
                        tbuf, [et_vecs[k], btl_vec, e8_vecs[k], b_vec], v
                    )

        pltpu.async_copy(
            tbuf.at[:, :, :8, :128],
            out_hbm.at[h, :, pl.ds(bt0, _BT), :, :], ssem,
        )

    start_gather(0, idx0, rows0, gsem0)

    @pl.loop(0, _NCHUNK, step=2)
    def _chunks(t):
        start_gather(t + 1, idx1, rows1, gsem1)
        transpose_store(t, idx0, rows0, gsem0, tbufA, ssemA, t >= 2)

        @pl.when(t + 2 < _NCHUNK)
        def _():
            start_gather(t + 2, idx0, rows0, gsem0)

        transpose_store(t + 1, idx1, rows1, gsem1, tbufB, ssemB, t >= 2)

    for last, tbuf, ssem in ((_NCHUNK - 2, tbufA, ssemA),
                             (_NCHUNK - 1, tbufB, ssemB)):
        lastu = ubase + last
        lh = lastu // (_BATCH // _CH)
        lbt = (lastu % (_BATCH // _CH)) * _BT
        pltpu.make_async_copy(
            tbuf.at[:, :, :8, :128],
            out_hbm.at[lh, :, pl.ds(lbt, _BT), :, :], ssem,
        ).wait()


def kernel(codes, table):
    flat = codes.T.reshape(-1).astype(jnp.int32)
    out5 = _gather_kernel(flat, table)
    return jnp.transpose(out5, (2, 4, 0, 1, 3)).reshape(_BATCH, _HIST, _EMBED)


# speedup vs baseline: 1.1144x; 1.1144x over previous
import functools

import jax
import jax.numpy as jnp
from jax import lax
from jax.experimental import pallas as pl
from jax.experimental.pallas import tpu as pltpu
from jax.experimental.pallas import tpu_sc as plsc

_BATCH = 16384
_HIST = 200
_EMBED = 64
_B = _BATCH * _HIST

_NC = 2
_NS = 16
_NW = _NC * _NS
_CH = 512
_NCHUNK = _B // (_NW * _CH)
_BT = _CH // 128
_CPH = _BATCH // _CH

_S_E8 = 129

_mesh = plsc.VectorSubcoreMesh(core_axis_name="c", subcore_axis_name="s")


@functools.partial(
    pl.kernel,
    out_type=jax.ShapeDtypeStruct((_HIST, 8, 128, 8, 128), jnp.float32),
    mesh=_mesh,
    scratch_types=[
        [pltpu.VMEM((_CH,), jnp.int32) for _ in range(4)],
        [pltpu.VMEM((_CH, _EMBED), jnp.float32) for _ in range(2)],
        pltpu.VMEM((8, _BT, 10, _S_E8), jnp.float32),
        [pltpu.SemaphoreType.DMA for _ in range(4)],
        [pltpu.SemaphoreType.DMA for _ in range(2)],
        pltpu.SemaphoreType.DMA,
    ],
    compiler_params=pltpu.CompilerParams(
        use_tc_tiling_on_sc=False, needs_layout_passes=False
    ),
)
def _gather_kernel(codes_hbm, table_hbm, out_hbm, idx, rows, tbuf,
                   isem, gsem, ssem):
    wid = lax.axis_index("s") * _NC + lax.axis_index("c")
    ubase = wid * _NCHUNK
    iota = lax.iota(jnp.int32, 16)
    et_vecs, e8_vecs = [], []
    for e0 in (0, 16, 32, 48):
        e_vec = e0 + iota
        et_vecs.append(e_vec >> 3)
        e8_vecs.append(e_vec & 7)

    def idx_slice(u):
        return codes_hbm.at[pl.ds((ubase + u) * _CH, _CH)]

    def out_slice(u):
        h = u // _CPH
        bt0 = (u % _CPH) * _BT
        return out_hbm.at[h, :, pl.ds(bt0, _BT), :, :]

    def substep(u, p2, p4):
        @pl.when(u + 2 < _NCHUNK)
        def _():
            pltpu.async_copy(idx_slice(u + 2), idx[(p4 + 2) % 4],
                             isem[(p4 + 2) % 4])

        @pl.when(u + 1 < _NCHUNK)
        def _():
            nx = (p4 + 1) % 4
            pltpu.make_async_copy(idx_slice(u + 1), idx[nx], isem[nx]).wait()
            pltpu.async_copy(table_hbm.at[idx[nx]], rows[(p2 + 1) % 2],
                             gsem[(p2 + 1) % 2])

        rows_v = rows[p2]
        pltpu.make_async_copy(table_hbm.at[idx[p4]], rows_v, gsem[p2]).wait()
        uu = ubase + u

        @pl.when(u >= 1)
        def _():
            pltpu.make_async_copy(
                tbuf.at[:, :, :8, :128], out_slice(uu - 1), ssem
            ).wait()

        for btl in range(_BT):
            btl_vec = jnp.full((16,), btl, jnp.int32)

            @plsc.parallel_loop(0, 128, unroll=4)
            def _row(b, btl=btl, btl_vec=btl_vec):
                r = btl * 128 + b
                b_vec = jnp.full((16,), b, jnp.int32)
                for k in range(4):
                    v = rows_v[r, pl.ds(k * 16, 16)]
                    plsc.store_scatter(
                        tbuf, [et_vecs[k], btl_vec, e8_vecs[k], b_vec], v
                    )

        pltpu.async_copy(tbuf.at[:, :, :8, :128], out_slice(uu), ssem)

    pltpu.sync_copy(idx_slice(0), idx[0])
    pltpu.async_copy(idx_slice(1), idx[1], isem[1])
    pltpu.async_copy(table_hbm.at[idx[0]], rows[0], gsem[0])

    @pl.loop(0, _NCHUNK, step=4)
    def _chunks(t):
        for s in range(4):
            substep(t + s, s % 2, s)

    lastu = ubase + _NCHUNK - 1
    pltpu.make_async_copy(
        tbuf.at[:, :, :8, :128], out_slice(lastu), ssem
    ).wait()


def kernel(codes, table):
    flat = codes.T.reshape(-1).astype(jnp.int32)
    out5 = _gather_kernel(flat, table)
    return jnp.transpose(out5, (2, 4, 0, 1, 3)).reshape(_BATCH, _HIST, _EMBED)
